# baseline (speedup 1.0000x reference)
import functools

import jax
import jax.numpy as jnp
from jax import lax
from jax.experimental import pallas as pl
from jax.experimental.pallas import tpu as pltpu
from jax.experimental.pallas import tpu_sc as plsc

B = 1024
P = 5
N = 10
DIM = 32
NPOS = P + N
TOT = (NPOS + 1) * B
GROUP = 128 // DIM

_NC = 2
_NS = 16
_NW = _NC * _NS
ROWS_PER_W = TOT // _NW
CHUNK = 128
NCHUNK = ROWS_PER_W // CHUNK


@functools.cache
def _make_sc_gather():
    @functools.partial(
        pl.kernel,
        out_type=jax.ShapeDtypeStruct((TOT, 128), jnp.float32),
        mesh=plsc.VectorSubcoreMesh(core_axis_name="c", subcore_axis_name="s"),
        scratch_types=[
            pltpu.VMEM((NCHUNK, CHUNK), jnp.int32),
            pltpu.VMEM((ROWS_PER_W, 128), jnp.float32),
            pltpu.SemaphoreType.DMA,
        ],
    )
    def _sc_gather(idx_hbm, table_hbm, out_hbm, idx_v, rows_v, sem):
        wid = lax.axis_index("s") * _NC + lax.axis_index("c")
        base = wid * ROWS_PER_W
        pltpu.sync_copy(idx_hbm.at[pl.ds(wid * NCHUNK, NCHUNK)], idx_v)
        copies = []
        for j in range(NCHUNK):
            copies.append(
                pltpu.async_copy(
                    table_hbm.at[idx_v.at[j]],
                    rows_v.at[pl.ds(j * CHUNK, CHUNK)],
                    sem,
                )
            )
        for c in copies:
            c.wait()
        pltpu.sync_copy(rows_v, out_hbm.at[pl.ds(base, ROWS_PER_W)])

    return _sc_gather


def _select(group_rows, offs):
    sel = jnp.zeros((B, DIM), jnp.float32)
    for k in range(GROUP):
        chunk = group_rows[:, k * DIM:(k + 1) * DIM]
        sel = sel + jnp.where(offs == k, chunk, 0.0)
    return sel


def _tc_body(center_ref, w_ref, oc_ref, ow_ref, out_ref):
    j = pl.program_id(0)
    sign = jnp.where(j < P, 1.0, -1.0)
    center = _select(center_ref[...], oc_ref[0])
    w = _select(w_ref[...], ow_ref[0])
    acc = lax.dot_general(
        center, w,
        (((1,), (1,)), ((), ())),
        preferred_element_type=jnp.float32,
    )
    out_ref[...] = jax.nn.sigmoid(sign * acc)


def _tc_cross(gathered, offs):
    return pl.pallas_call(
        _tc_body,
        grid=(NPOS,),
        in_specs=[
            pl.BlockSpec((B, 128), lambda j: (NPOS, 0)),
            pl.BlockSpec((B, 128), lambda j: (j, 0)),
            pl.BlockSpec((1, B, 1), lambda j: (NPOS, 0, 0)),
            pl.BlockSpec((1, B, 1), lambda j: (j, 0, 0)),
        ],
        out_specs=pl.BlockSpec((B, B), lambda j: (0, j)),
        out_shape=jax.ShapeDtypeStruct((B, NPOS * B), jnp.float32),
    )(gathered, gathered, offs, offs)


def kernel(x_center, x_positive, x_negative, emb_table):
    idx_all = jnp.concatenate([
        x_positive.T.reshape(-1),
        x_negative.T.reshape(-1),
        x_center,
    ]).astype(jnp.int32)
    group_idx = (idx_all // GROUP).reshape(TOT // CHUNK, CHUNK)
    offs = (idx_all % GROUP).reshape(NPOS + 1, B, 1)
    table_g = emb_table.reshape(emb_table.shape[0] // GROUP, 128)
    gathered = _make_sc_gather()(group_idx, table_g)
    return _tc_cross(gathered, offs)


# speedup vs baseline: 3.2693x; 3.2693x over previous
import functools

import jax
import jax.numpy as jnp
from jax import lax
from jax.experimental import pallas as pl
from jax.experimental.pallas import tpu as pltpu
from jax.experimental.pallas import tpu_sc as plsc

B = 1024
P = 5
N = 10
DIM = 32
NPOS = P + N
TOT = (NPOS + 1) * B

_NC = 2
_NS = 16
_NW = _NC * _NS
ROWS_PER_W = TOT // _NW
BATCH = 16
NBATCH = ROWS_PER_W // BATCH


@functools.cache
def _make_sc_gather():
    @functools.partial(
        pl.kernel,
        out_type=jax.ShapeDtypeStruct((TOT, DIM), jnp.float32),
        mesh=plsc.VectorSubcoreMesh(core_axis_name="c", subcore_axis_name="s"),
        scratch_types=[
            pltpu.VMEM((ROWS_PER_W,), jnp.int32),
            pltpu.VMEM((BATCH, DIM, 128), jnp.float32),
            pltpu.VMEM((BATCH, DIM), jnp.float32),
            pltpu.SemaphoreType.DMA,
        ],
        compiler_params=pltpu.CompilerParams(
            use_tc_tiling_on_sc=True, needs_layout_passes=False
        ),
    )
    def _sc_gather(idx_hbm, tableT_hbm, out_hbm, idx_v, slab_v, rows_v,
                   sem):
        wid = lax.axis_index("s") * _NC + lax.axis_index("c")
        base = wid * ROWS_PER_W
        pltpu.sync_copy(idx_hbm.at[pl.ds(base, ROWS_PER_W)], idx_v)

        rows16 = lax.iota(jnp.int32, 16)

        def batch_body(b, carry):
            idvec = idx_v[pl.ds(b * BATCH, BATCH)]
            copies = []
            ids = []
            for k in range(BATCH):
                idx = idvec[k]
                ids.append(idx)
                l0 = (idx // 128) * 128
                copies.append(
                    pltpu.async_copy(
                        tableT_hbm.at[:, pl.ds(l0, 128)], slab_v.at[k], sem
                    )
                )
            for c in copies:
                c.wait()
            for k in range(BATCH):
                colv = jnp.full((16,), ids[k] % 128, jnp.int32)
                v0 = plsc.load_gather(slab_v.at[k], [rows16, colv])
                v1 = plsc.load_gather(slab_v.at[k], [rows16 + 16, colv])
                rows_v[k, pl.ds(0, 16)] = v0
                rows_v[k, pl.ds(16, 16)] = v1
            pltpu.sync_copy(
                rows_v, out_hbm.at[pl.ds(base + b * BATCH, BATCH)]
            )
            return carry

        lax.fori_loop(0, NBATCH, batch_body, 0)

    return _sc_gather


def _tc_body(center_ref, w_ref, out_ref):
    j = pl.program_id(0)
    sign = jnp.where(j < P, 1.0, -1.0)
    acc = lax.dot_general(
        center_ref[...], w_ref[...],
        (((1,), (1,)), ((), ())),
        preferred_element_type=jnp.float32,
    )
    out_ref[...] = jax.nn.sigmoid(sign * acc)


def _tc_cross(gathered):
    return pl.pallas_call(
        _tc_body,
        grid=(NPOS,),
        in_specs=[
            pl.BlockSpec((B, DIM), lambda j: (NPOS, 0)),
            pl.BlockSpec((B, DIM), lambda j: (j, 0)),
        ],
        out_specs=pl.BlockSpec((B, B), lambda j: (0, j)),
        out_shape=jax.ShapeDtypeStruct((B, NPOS * B), jnp.float32),
    )(gathered, gathered)


def kernel(x_center, x_positive, x_negative, emb_table):
    idx_all = jnp.concatenate([
        x_positive.T.reshape(-1),
        x_negative.T.reshape(-1),
        x_center,
    ]).astype(jnp.int32)
    gathered = _make_sc_gather()(idx_all, emb_table.T)
    return _tc_cross(gathered)


# speedup vs baseline: 3.4366x; 1.0512x over previous
import functools

import jax
import jax.numpy as jnp
from jax import lax
from jax.experimental import pallas as pl
from jax.experimental.pallas import tpu as pltpu
from jax.experimental.pallas import tpu_sc as plsc

B = 1024
P = 5
N = 10
DIM = 32
NPOS = P + N
TOT = (NPOS + 1) * B

_NC = 2
_NS = 16
_NW = _NC * _NS
ROWS_PER_W = TOT // _NW
BATCH = 8
NBATCH = ROWS_PER_W // BATCH


@functools.cache
def _make_sc_gather():
    @functools.partial(
        pl.kernel,
        out_type=jax.ShapeDtypeStruct((TOT, DIM), jnp.float32),
        mesh=plsc.VectorSubcoreMesh(core_axis_name="c", subcore_axis_name="s"),
        scratch_types=[
            pltpu.VMEM((ROWS_PER_W,), jnp.int32),
            pltpu.VMEM((2, BATCH, DIM, 128), jnp.float32),
            pltpu.VMEM((BATCH, DIM), jnp.float32),
            pltpu.SemaphoreType.DMA,
            pltpu.SemaphoreType.DMA,
        ],
        compiler_params=pltpu.CompilerParams(
            use_tc_tiling_on_sc=True, needs_layout_passes=False
        ),
    )
    def _sc_gather(idx_hbm, tableT_hbm, out_hbm, idx_v, slab_v, rows_v,
                   sem_a, sem_b):
        wid = lax.axis_index("s") * _NC + lax.axis_index("c")
        base = wid * ROWS_PER_W
        pltpu.sync_copy(idx_hbm.at[pl.ds(base, ROWS_PER_W)], idx_v)

        rows16 = lax.iota(jnp.int32, 16)

        def fire(b, bank, sem):
            idvec = idx_v[pl.ds((b // 2) * (2 * BATCH), 2 * BATCH)]
            for k in range(BATCH):
                l0 = (idvec[bank * BATCH + k] // 128) * 128
                pltpu.async_copy(
                    tableT_hbm.at[:, pl.ds(l0, 128)], slab_v.at[bank, k], sem
                )

        def drain_extract_flush(b, bank, sem):
            for k in range(BATCH):
                pltpu.make_async_copy(
                    tableT_hbm.at[:, pl.ds(0, 128)], slab_v.at[bank, k], sem
                ).wait()
            idvec = idx_v[pl.ds((b // 2) * (2 * BATCH), 2 * BATCH)]
            for k in range(BATCH):
                colv = jnp.full((16,), idvec[bank * BATCH + k] % 128, jnp.int32)
                v0 = plsc.load_gather(slab_v.at[bank, k], [rows16, colv])
                v1 = plsc.load_gather(slab_v.at[bank, k], [rows16 + 16, colv])
                rows_v[k, pl.ds(0, 16)] = v0
                rows_v[k, pl.ds(16, 16)] = v1
            pltpu.sync_copy(
                rows_v, out_hbm.at[pl.ds(base + b * BATCH, BATCH)]
            )

        fire(0, 0, sem_a)

        def body(i, carry):
            fire(2 * i + 1, 1, sem_b)
            drain_extract_flush(2 * i, 0, sem_a)
            fire(2 * i + 2, 0, sem_a)
            drain_extract_flush(2 * i + 1, 1, sem_b)
            return carry

        lax.fori_loop(0, NBATCH // 2 - 1, body, 0)
        fire(NBATCH - 1, 1, sem_b)
        drain_extract_flush(NBATCH - 2, 0, sem_a)
        drain_extract_flush(NBATCH - 1, 1, sem_b)

    return _sc_gather


def _tc_body(center_ref, w_ref, out_ref):
    j = pl.program_id(0)
    sign = jnp.where(j < P, 1.0, -1.0)
    acc = lax.dot_general(
        center_ref[...], w_ref[...],
        (((1,), (1,)), ((), ())),
        preferred_element_type=jnp.float32,
    )
    out_ref[...] = jax.nn.sigmoid(sign * acc)


def _tc_cross(gathered):
    return pl.pallas_call(
        _tc_body,
        grid=(NPOS,),
        in_specs=[
            pl.BlockSpec((B, DIM), lambda j: (NPOS, 0)),
            pl.BlockSpec((B, DIM), lambda j: (j, 0)),
        ],
        out_specs=pl.BlockSpec((B, B), lambda j: (0, j)),
        out_shape=jax.ShapeDtypeStruct((B, NPOS * B), jnp.float32),
    )(gathered, gathered)


def kernel(x_center, x_positive, x_negative, emb_table):
    idx_all = jnp.concatenate([
        x_positive.T.reshape(-1),
        x_negative.T.reshape(-1),
        x_center,
    ]).astype(jnp.int32)
    gathered = _make_sc_gather()(idx_all, emb_table.T)
    return _tc_cross(gathered)
